# baseline (speedup 1.0000x reference)
import numpy as np
import jax
import jax.numpy as jnp
from jax.experimental import pallas as pl
from jax.experimental.pallas import tpu as pltpu

_B = 256
_C_IN = 3
_H = _W = 96
_PATCH = 4
_HP = _WP = 24
_P = _HP * _WP
_CP = _C_IN * _PATCH * _PATCH
_Q = 16
_K = 6
_HW = _H * _W
_BBLK = 16
_NB = _B // _BBLK
_HWT = 9216
_NHT = _HW // _HWT


def _bilinear_matrix(out_size, in_size):
    scale = in_size / out_size
    A = np.zeros((out_size, in_size), np.float32)
    for i in range(out_size):
        src = max((i + 0.5) * scale - 0.5, 0.0)
        i0 = min(int(np.floor(src)), in_size - 1)
        i1 = min(i0 + 1, in_size - 1)
        w = src - i0
        A[i, i0] += 1.0 - w
        A[i, i1] += w
    return A


def _seg_kernel(pt_ref, qw_ref, mb_ref, u_ref, qe_ref, wc_ref, cb_ref, o_ref):
    logits = (
        jnp.dot(qe_ref[...], wc_ref[...], preferred_element_type=jnp.float32)
        + cb_ref[...]
    )
    mx = jnp.max(logits, axis=-1, keepdims=True)
    e = jnp.exp(logits - mx)
    p = e / jnp.sum(e, axis=-1, keepdims=True)
    pT = jnp.tile(p[:, :_K].T, (_BBLK, _BBLK))
    rb = jax.lax.broadcasted_iota(jnp.int32, pT.shape, 0) // _K
    cc = jax.lax.broadcasted_iota(jnp.int32, pT.shape, 1) // _Q
    pb = jnp.where(rb == cc, pT, 0.0).astype(jnp.bfloat16)
    p3 = pt_ref[...].reshape(_BBLK, _P, _CP)
    pt = jnp.transpose(p3, (0, 2, 1)).reshape(_BBLK * _CP, _P)
    mlog = (
        jnp.dot(qw_ref[...], pt, preferred_element_type=jnp.float32)
        + mb_ref[...]
    ).astype(jnp.bfloat16)
    for t in range(_NHT):
        sl = pl.ds(t * _HWT, _HWT)
        up = jnp.dot(mlog, u_ref[:, sl], preferred_element_type=jnp.float32)
        sig = jax.nn.sigmoid(up).astype(jnp.bfloat16)
        red = jnp.dot(pb, sig, preferred_element_type=jnp.float32)
        o_ref[:, :, sl] = red.reshape(_BBLK, _K, _HWT)


def kernel(image, pixel_mean, pixel_std, w_patch, b_patch, w_pix, b_pix,
           query_embed, w_cls, b_cls):
    f32 = jnp.float32
    bf16 = jnp.bfloat16

    inv_std = (1.0 / pixel_std).astype(f32)
    scale = jnp.repeat(inv_std, _PATCH * _PATCH)
    shift = jnp.repeat(pixel_mean.astype(f32) * inv_std, _PATCH * _PATCH)
    w_scaled = scale[:, None] * w_patch.astype(f32)
    b_scaled = b_patch.astype(f32) - shift @ w_patch.astype(f32)
    w_f = w_scaled @ w_pix.astype(f32)
    b_f = b_scaled @ w_pix.astype(f32) + b_pix.astype(f32)

    qe = query_embed.astype(f32)
    qa = qe @ w_f.T
    abias = qe @ b_f
    eye = jnp.eye(_BBLK, dtype=f32)
    qw = jnp.kron(eye, qa).astype(bf16)
    mb = jnp.tile(abias, _BBLK).reshape(_BBLK * _Q, 1)

    Ah = _bilinear_matrix(_H, _HP)
    Aw = _bilinear_matrix(_W, _WP)
    u = jnp.asarray(np.kron(Ah, Aw).T).astype(bf16)

    patches = image.astype(bf16).reshape(_B, _C_IN, _HP, _PATCH, _WP, _PATCH)
    patches = patches.transpose(0, 2, 4, 1, 3, 5).reshape(_B, _P * _CP)

    out = pl.pallas_call(
        _seg_kernel,
        out_shape=jax.ShapeDtypeStruct((_B, _K, _HW), f32),
        grid=(_NB,),
        in_specs=[
            pl.BlockSpec((_BBLK, _P * _CP), lambda i: (i, 0)),
            pl.BlockSpec((_BBLK * _Q, _BBLK * _CP), lambda i: (0, 0)),
            pl.BlockSpec((_BBLK * _Q, 1), lambda i: (0, 0)),
            pl.BlockSpec((_P, _HW), lambda i: (0, 0)),
            pl.BlockSpec((_Q, 32), lambda i: (0, 0)),
            pl.BlockSpec((32, _K + 1), lambda i: (0, 0)),
            pl.BlockSpec((1, _K + 1), lambda i: (0, 0)),
        ],
        out_specs=pl.BlockSpec((_BBLK, _K, _HW), lambda i: (i, 0, 0)),
        compiler_params=pltpu.CompilerParams(
            dimension_semantics=("parallel",)
        ),
    )(patches, qw, mb, u, qe, w_cls.astype(f32),
      b_cls.reshape(1, -1).astype(f32))

    return out.reshape(_B, _K, _H, _W)


# speedup vs baseline: 1.2162x; 1.2162x over previous
import numpy as np
import jax
import jax.numpy as jnp
from jax.experimental import pallas as pl
from jax.experimental.pallas import tpu as pltpu

_B = 256
_C_IN = 3
_H = _W = 96
_PATCH = 4
_HP = _WP = 24
_P = _HP * _WP
_CP = _C_IN * _PATCH * _PATCH
_Q = 16
_K = 6
_HW = _H * _W
_BBLK = 16
_NB = _B // _BBLK
_HWT = 9216
_NHT = _HW // _HWT


def _bilinear_matrix(out_size, in_size):
    scale = in_size / out_size
    A = np.zeros((out_size, in_size), np.float32)
    for i in range(out_size):
        src = max((i + 0.5) * scale - 0.5, 0.0)
        i0 = min(int(np.floor(src)), in_size - 1)
        i1 = min(i0 + 1, in_size - 1)
        w = src - i0
        A[i, i0] += 1.0 - w
        A[i, i1] += w
    return A


def _seg_kernel(pt_ref, qw_ref, mb_ref, u_ref, qe_ref, wc_ref, cb_ref, o_ref):
    logits = (
        jnp.dot(qe_ref[...], wc_ref[...], preferred_element_type=jnp.float32)
        + cb_ref[...]
    )
    mx = jnp.max(logits, axis=-1, keepdims=True)
    e = jnp.exp(logits - mx)
    p = e / jnp.sum(e, axis=-1, keepdims=True)
    pT = jnp.tile(p[:, :_K].T, (_BBLK, _BBLK))
    rb = jax.lax.broadcasted_iota(jnp.int32, pT.shape, 0) // _K
    cc = jax.lax.broadcasted_iota(jnp.int32, pT.shape, 1) // _Q
    pb = jnp.where(rb == cc, pT, 0.0).astype(jnp.bfloat16)
    mlog = (
        jnp.dot(qw_ref[...], pt_ref[0], preferred_element_type=jnp.float32)
        + mb_ref[...]
    ).astype(jnp.bfloat16)
    for t in range(_NHT):
        sl = pl.ds(t * _HWT, _HWT)
        up = jnp.dot(mlog, u_ref[:, sl], preferred_element_type=jnp.float32)
        sig = jax.nn.sigmoid(up).astype(jnp.bfloat16)
        red = jnp.dot(pb, sig, preferred_element_type=jnp.float32)
        o_ref[:, :, sl] = red.reshape(_BBLK, _K, _HWT)


def kernel(image, pixel_mean, pixel_std, w_patch, b_patch, w_pix, b_pix,
           query_embed, w_cls, b_cls):
    f32 = jnp.float32
    bf16 = jnp.bfloat16

    inv_std = (1.0 / pixel_std).astype(f32)
    scale = jnp.repeat(inv_std, _PATCH * _PATCH)
    shift = jnp.repeat(pixel_mean.astype(f32) * inv_std, _PATCH * _PATCH)
    w_scaled = scale[:, None] * w_patch.astype(f32)
    b_scaled = b_patch.astype(f32) - shift @ w_patch.astype(f32)
    w_f = w_scaled @ w_pix.astype(f32)
    b_f = b_scaled @ w_pix.astype(f32) + b_pix.astype(f32)

    qe = query_embed.astype(f32)
    qa = qe @ w_f.T
    abias = qe @ b_f
    eye = jnp.eye(_BBLK, dtype=f32)
    qw = jnp.kron(eye, qa).astype(bf16)
    mb = jnp.tile(abias, _BBLK).reshape(_BBLK * _Q, 1)

    Ah = _bilinear_matrix(_H, _HP)
    Aw = _bilinear_matrix(_W, _WP)
    u = jnp.asarray(np.kron(Ah, Aw).T).astype(bf16)

    patches = image.astype(bf16).reshape(_B, _C_IN, _HP, _PATCH, _WP, _PATCH)
    patches = patches.transpose(0, 2, 4, 1, 3, 5).reshape(_B, _P, _CP)
    patches = jnp.swapaxes(patches, 1, 2)
    patches = patches.reshape(_NB, _BBLK * _CP, _P)

    out = pl.pallas_call(
        _seg_kernel,
        out_shape=jax.ShapeDtypeStruct((_B, _K, _HW), f32),
        grid=(_NB,),
        in_specs=[
            pl.BlockSpec((1, _BBLK * _CP, _P), lambda i: (i, 0, 0)),
            pl.BlockSpec((_BBLK * _Q, _BBLK * _CP), lambda i: (0, 0)),
            pl.BlockSpec((_BBLK * _Q, 1), lambda i: (0, 0)),
            pl.BlockSpec((_P, _HW), lambda i: (0, 0)),
            pl.BlockSpec((_Q, 32), lambda i: (0, 0)),
            pl.BlockSpec((32, _K + 1), lambda i: (0, 0)),
            pl.BlockSpec((1, _K + 1), lambda i: (0, 0)),
        ],
        out_specs=pl.BlockSpec((_BBLK, _K, _HW), lambda i: (i, 0, 0)),
        compiler_params=pltpu.CompilerParams(
            dimension_semantics=("parallel",)
        ),
    )(patches, qw, mb, u, qe, w_cls.astype(f32),
      b_cls.reshape(1, -1).astype(f32))

    return out.reshape(_B, _K, _H, _W)
